# baseline (speedup 1.0000x reference)
import functools

import jax
import jax.numpy as jnp
from jax import lax
from jax.experimental import pallas as pl
from jax.experimental.pallas import tpu as pltpu
from jax.experimental.pallas import tpu_sc as plsc

D0, D1, D2 = 15, 16, 12
NT = D0 * D1 * D2
EMB = 64
NC, NS, L = 2, 16, 16
NW = NC * NS
P = 512
G = 128
IB = 8


def _table_body(w0_ref, w1_ref, w2_ref, t_ref):
    w0 = w0_ref[...]
    w1 = w1_ref[...]
    w2 = w2_ref[...]
    t_ref[...] = (w0[:, None, None, :] + w1[None, :, None, :]
                  + w2[None, None, :, :])


def _build_table(w0, w1, w2):
    t4 = pl.pallas_call(
        _table_body,
        out_shape=jax.ShapeDtypeStruct((D0, D1, D2, EMB), jnp.float32),
    )(w0, w1, w2)
    return t4.reshape(NT, EMB)


def _cidx_body(e_ref, c_ref):
    x = e_ref[...]
    c = x[..., 0] * (D1 * D2) + x[..., 1] * D2 + x[..., 2]
    c_ref[...] = c.reshape(c_ref.shape)


def _build_cidx(edge_attr):
    B, I, J, K, _ = edge_attr.shape
    n_rows = B * I * J * K
    rows_per_blk = IB * J * K // 128
    return pl.pallas_call(
        _cidx_body,
        grid=(B, I // IB),
        in_specs=[pl.BlockSpec((1, IB, J, K, 3), lambda b, i: (b, i, 0, 0, 0))],
        out_specs=pl.BlockSpec((rows_per_blk, 128),
                               lambda b, i: (b * (I // IB) + i, 0)),
        out_shape=jax.ShapeDtypeStruct((n_rows // 128, 128), jnp.int32),
    )(edge_attr)


def _make_gather(n_rows):
    npw = n_rows // NW
    steps = npw // P
    half = steps // 2
    n_g = P // G
    mesh = plsc.VectorSubcoreMesh(core_axis_name="c", subcore_axis_name="s")

    @functools.partial(
        pl.kernel,
        mesh=mesh,
        compiler_params=pltpu.CompilerParams(
            needs_layout_passes=False, use_tc_tiling_on_sc=False),
        out_type=jax.ShapeDtypeStruct((n_rows, EMB), jnp.float32),
        scratch_types=[
            pltpu.VMEM_SHARED((NT, EMB), jnp.float32),
            pltpu.VMEM((n_g, G), jnp.int32),
            pltpu.VMEM((n_g, G), jnp.int32),
            pltpu.VMEM((P, EMB), jnp.float32),
            pltpu.VMEM((P, EMB), jnp.float32),
            pltpu.SemaphoreType.DMA,
            pltpu.SemaphoreType.DMA,
            pltpu.SemaphoreType.DMA,
            pltpu.SemaphoreType.DMA,
        ],
    )
    def k(c_hbm, t_hbm, out_hbm, t_sp, ci_v0, ci_v1, r_v0, r_v1,
          g_s0, g_s1, s_s0, s_s1):
        sid = lax.axis_index("s")
        wid = sid * NC + lax.axis_index("c")

        @pl.when(sid == 0)
        def _():
            pltpu.sync_copy(t_hbm, t_sp)
        plsc.subcore_barrier()

        def fire(i, cidx_v, rows_v, gsem):
            base = (wid * npw + i * P) // G
            pltpu.sync_copy(c_hbm.at[pl.ds(base, n_g)], cidx_v)
            return [
                pltpu.async_copy(
                    t_sp.at[cidx_v.at[g]],
                    rows_v.at[pl.ds(g * G, G)],
                    gsem,
                )
                for g in range(n_g)
            ]

        def wait_store(rows_v, ssem):
            pltpu.make_async_copy(rows_v, out_hbm.at[pl.ds(0, P)], ssem).wait()

        def body(h, carry):
            i0, i1 = 2 * h, 2 * h + 1

            @pl.when(h > 0)
            def _():
                wait_store(r_v0, s_s0)
                wait_store(r_v1, s_s1)

            cps0 = fire(i0, ci_v0, r_v0, g_s0)
            cps1 = fire(i1, ci_v1, r_v1, g_s1)
            for cp in cps0:
                cp.wait()
            pltpu.async_copy(
                r_v0, out_hbm.at[pl.ds(wid * npw + i0 * P, P)], s_s0)
            for cp in cps1:
                cp.wait()
            pltpu.async_copy(
                r_v1, out_hbm.at[pl.ds(wid * npw + i1 * P, P)], s_s1)
            return carry

        lax.fori_loop(0, half, body, 0)
        wait_store(r_v0, s_s0)
        wait_store(r_v1, s_s1)

    return k


def kernel(edge_attr, w0, w1, w2):
    shp = edge_attr.shape
    n_rows = edge_attr.size // 3
    t = _build_table(w0, w1, w2)
    cidx = _build_cidx(edge_attr.astype(jnp.int32))
    out = _make_gather(n_rows)(cidx, t)
    return out.reshape(*shp[:-1], EMB)


# speedup vs baseline: 1.1649x; 1.1649x over previous
import functools

import jax
import jax.numpy as jnp
from jax import lax
from jax.experimental import pallas as pl
from jax.experimental.pallas import tpu as pltpu
from jax.experimental.pallas import tpu_sc as plsc

D0, D1, D2 = 15, 16, 12
NT = D0 * D1 * D2
EMB = 64
NC, NS, L = 2, 16, 16
NW = NC * NS
P = 512
G = 128


def _table_body(w0_ref, w1_ref, w2_ref, t_ref):
    w0 = w0_ref[...]
    w1 = w1_ref[...]
    w2 = w2_ref[...]
    t_ref[...] = (w0[:, None, None, :] + w1[None, :, None, :]
                  + w2[None, None, :, :])


def _build_table(w0, w1, w2):
    t4 = pl.pallas_call(
        _table_body,
        out_shape=jax.ShapeDtypeStruct((D0, D1, D2, EMB), jnp.float32),
    )(w0, w1, w2)
    return t4.reshape(NT, EMB)


def _make_gather(n_rows):
    npw = n_rows // NW
    steps = npw // P
    half = steps // 2
    erows = 3 * P // 128
    mesh = plsc.VectorSubcoreMesh(core_axis_name="c", subcore_axis_name="s")

    @functools.partial(
        pl.kernel,
        mesh=mesh,
        compiler_params=pltpu.CompilerParams(
            needs_layout_passes=False, use_tc_tiling_on_sc=False),
        out_type=jax.ShapeDtypeStruct((n_rows, EMB), jnp.float32),
        scratch_types=[
            pltpu.VMEM_SHARED((NT, EMB), jnp.float32),
            pltpu.VMEM((erows, 128), jnp.int32),
            pltpu.VMEM((erows, 128), jnp.int32),
            pltpu.VMEM((P // G, G), jnp.int32),
            pltpu.VMEM((P // G, G), jnp.int32),
            pltpu.VMEM((P, EMB), jnp.float32),
            pltpu.VMEM((P, EMB), jnp.float32),
            pltpu.SemaphoreType.DMA,
            pltpu.SemaphoreType.DMA,
            pltpu.SemaphoreType.DMA,
            pltpu.SemaphoreType.DMA,
        ],
    )
    def k(e_hbm, t_hbm, out_hbm, t_sp, e_v0, e_v1, ci_v0, ci_v1, r_v0, r_v1,
          g_s0, g_s1, s_s0, s_s1):
        sid = lax.axis_index("s")
        wid = sid * NC + lax.axis_index("c")
        iota = lax.iota(jnp.int32, L)

        @pl.when(sid == 0)
        def _():
            pltpu.sync_copy(t_hbm, t_sp)
        plsc.subcore_barrier()

        def fire(i, e_v, cidx_v, rows_v, gsem):
            base = wid * npw + i * P
            pltpu.sync_copy(e_hbm.at[pl.ds(3 * base // 128, erows)], e_v)
            for j in range(P // L):
                flat = iota * 3 + (3 * L * j)
                e0 = plsc.load_gather(e_v, [flat // 128, lax.rem(flat, 128)])
                f1 = flat + 1
                e1 = plsc.load_gather(e_v, [f1 // 128, lax.rem(f1, 128)])
                f2 = flat + 2
                e2 = plsc.load_gather(e_v, [f2 // 128, lax.rem(f2, 128)])
                c = e0 * (D1 * D2) + e1 * D2 + e2
                cidx_v[(j * L) // G, pl.ds((j * L) % G, L)] = c
            return [
                pltpu.async_copy(
                    t_sp.at[cidx_v.at[g]],
                    rows_v.at[pl.ds(g * G, G)],
                    gsem,
                )
                for g in range(P // G)
            ]

        def wait_store(rows_v, ssem):
            pltpu.make_async_copy(rows_v, out_hbm.at[pl.ds(0, P)], ssem).wait()

        def body(h, carry):
            i0, i1 = 2 * h, 2 * h + 1

            @pl.when(h > 0)
            def _():
                wait_store(r_v0, s_s0)
                wait_store(r_v1, s_s1)

            cps0 = fire(i0, e_v0, ci_v0, r_v0, g_s0)
            cps1 = fire(i1, e_v1, ci_v1, r_v1, g_s1)
            for cp in cps0:
                cp.wait()
            pltpu.async_copy(
                r_v0, out_hbm.at[pl.ds(wid * npw + i0 * P, P)], s_s0)
            for cp in cps1:
                cp.wait()
            pltpu.async_copy(
                r_v1, out_hbm.at[pl.ds(wid * npw + i1 * P, P)], s_s1)
            return carry

        lax.fori_loop(0, half, body, 0)
        wait_store(r_v0, s_s0)
        wait_store(r_v1, s_s1)

    return k


def kernel(edge_attr, w0, w1, w2):
    shp = edge_attr.shape
    n_rows = edge_attr.size // 3
    e2d = edge_attr.reshape(3 * n_rows // 128, 128)
    t = _build_table(w0, w1, w2)
    out = _make_gather(n_rows)(e2d, t)
    return out.reshape(*shp[:-1], EMB)
